# Initial kernel scaffold; baseline (speedup 1.0000x reference)
import jax
import jax.numpy as jnp
from jax.experimental import pallas as pl


def kernel(x, edge_index, adj, link_weight, W_att, a_att, gcn1_w, gcn1_b, gcn2_w, gcn2_b):
    raise NotImplementedError("write your pallas kernel here")


# speedup vs baseline: 1.1158x; 1.1158x over previous
import functools
import jax
import jax.numpy as jnp
from jax import lax
from jax.experimental import pallas as pl
from jax.experimental.pallas import tpu as pltpu

N = 4096
E = 65536
IN = 128
HID = 8
HEADS = 8
OUT = 64

BR = 256
NB = N // BR


def _pre_body(x_ref, w2_ref, a1_ref, a2_ref, wh_ref, e1_ref, e2t_ref):
    wh = jnp.dot(x_ref[...], w2_ref[...], preferred_element_type=jnp.float32)
    wh_ref[...] = wh
    e1_ref[...] = jnp.dot(wh, a1_ref[...], preferred_element_type=jnp.float32)
    e2t_ref[...] = lax.dot_general(
        a2_ref[...], wh, (((0,), (1,)), ((), ())),
        preferred_element_type=jnp.float32)


def _gat_body(adj_ref, lw_ref, e1_ref, e2t_ref, wh_ref, out_ref):
    adjv = adj_ref[...]
    lwv = lw_ref[...]
    for h in range(HEADS):
        lg = e1_ref[:, h:h + 1] + e2t_ref[h:h + 1, :]
        lg = jnp.maximum(lg, 0.2 * lg)
        lg = lg * lwv
        lg = jnp.where(adjv > 0, lg, jnp.float32(-9e15))
        m = jnp.max(lg, axis=1, keepdims=True)
        p = jnp.exp(lg - m)
        s = jnp.sum(p, axis=1, keepdims=True)
        att = p / s
        o = jnp.dot(att, wh_ref[:, h * HID:(h + 1) * HID],
                    preferred_element_type=jnp.float32)
        out_ref[:, h * HID:(h + 1) * HID] = jnp.where(o > 0, o, jnp.exp(o) - 1.0)


@jax.jit
def _gat(x, adj, lw, W_att, a_att):
    w2 = W_att.transpose(1, 0, 2).reshape(IN, HEADS * HID)
    a1 = a_att[:, :HID, 0]
    a2 = a_att[:, HID:, 0]
    eye = jnp.eye(HEADS, dtype=jnp.float32)
    A1 = (a1[:, :, None] * eye[:, None, :]).reshape(HEADS * HID, HEADS)
    A2 = (a2[:, :, None] * eye[:, None, :]).reshape(HEADS * HID, HEADS)

    wh, e1, e2t = pl.pallas_call(
        _pre_body,
        out_shape=[
            jax.ShapeDtypeStruct((N, HEADS * HID), jnp.float32),
            jax.ShapeDtypeStruct((N, HEADS), jnp.float32),
            jax.ShapeDtypeStruct((HEADS, N), jnp.float32),
        ],
    )(x, w2, A1, A2)

    h_out = pl.pallas_call(
        _gat_body,
        grid=(NB,),
        in_specs=[
            pl.BlockSpec((BR, N), lambda i: (i, 0)),
            pl.BlockSpec((BR, N), lambda i: (i, 0)),
            pl.BlockSpec((BR, HEADS), lambda i: (i, 0)),
            pl.BlockSpec((HEADS, N), lambda i: (0, 0)),
            pl.BlockSpec((N, HEADS * HID), lambda i: (0, 0)),
        ],
        out_specs=pl.BlockSpec((BR, HEADS * HID), lambda i: (i, 0)),
        out_shape=jax.ShapeDtypeStruct((N, HEADS * HID), jnp.float32),
    )(adj, lw, e1, e2t, wh)
    return h_out


def _gcn_jnp(h, src, dst, W, b):
    loop = jnp.arange(N, dtype=src.dtype)
    s = jnp.concatenate([src, loop])
    d = jnp.concatenate([dst, loop])
    deg = jnp.zeros((N,), jnp.float32).at[d].add(1.0)
    dinv = lax.rsqrt(jnp.maximum(deg, 1.0))
    norm = dinv[s] * dinv[d]
    xw = h @ W
    out = jax.ops.segment_sum(xw[s] * norm[:, None], d, num_segments=N)
    return out + b


def kernel(x, edge_index, adj, link_weight, W_att, a_att,
           gcn1_w, gcn1_b, gcn2_w, gcn2_b):
    h = _gat(x, adj, link_weight, W_att, a_att)
    src, dst = edge_index[0], edge_index[1]
    h = _gcn_jnp(h, src, dst, gcn1_w, gcn1_b)
    h = jax.nn.relu(h)
    z = _gcn_jnp(h, src, dst, gcn2_w, gcn2_b)
    return z
